# Initial kernel scaffold; baseline (speedup 1.0000x reference)
import jax
import jax.numpy as jnp
from jax.experimental import pallas as pl


def kernel(x_s, x_t, edge_index, edge_attr, u, batch_e, W1, b1, W2, b2):
    raise NotImplementedError("write your pallas kernel here")


# speedup vs baseline: 7.6870x; 7.6870x over previous
import functools

import jax
import jax.numpy as jnp
from jax import lax
from jax.experimental import pallas as pl
from jax.experimental.pallas import tpu as pltpu
from jax.experimental.pallas import tpu_sc as plsc

D16 = 16
NC = 2
NS = 16
NW = NC * NS

C = 1024
GR = 128
SUB = C // GR

BE = 2048


def _prep_body(xst_ref, xtt_ref, ut_ref, w1s_ref, w1t_ref, w1u_ref, b1_ref,
               pst_ref, ptt_ref, put_ref):
    dims = (((0,), (0,)), ((), ()))
    pst_ref[...] = lax.dot_general(w1s_ref[...], xst_ref[...], dims,
                                   preferred_element_type=jnp.float32)
    ptt_ref[...] = lax.dot_general(w1t_ref[...], xtt_ref[...], dims,
                                   preferred_element_type=jnp.float32)
    put_ref[...] = lax.dot_general(w1u_ref[...], ut_ref[...], dims,
                                   preferred_element_type=jnp.float32
                                   ) + b1_ref[...]


def _finish_body(*refs):
    g_refs = refs[:D16]
    at_ref, brow_ref, put_ref, w1e_ref, w2_ref, b2_ref, out_ref = refs[D16:]
    be = at_ref.shape[1]
    nb = put_ref.shape[1]
    dims = (((0,), (0,)), ((), ()))
    a_term = lax.dot_general(w1e_ref[...], at_ref[...], dims,
                             preferred_element_type=jnp.float32)
    onehot = (brow_ref[...] == lax.broadcasted_iota(jnp.int32, (nb, be), 0)
              ).astype(jnp.float32)
    u_term = lax.dot_general(put_ref[...], onehot,
                             (((1,), (0,)), ((), ())),
                             preferred_element_type=jnp.float32)
    g = jnp.concatenate([r[...].reshape(1, be) for r in g_refs], axis=0)
    h = g + a_term + u_term
    h = jnp.where(h >= 0, h, 0.1 * h)
    out_ref[...] = lax.dot_general(w2_ref[...], h, dims,
                                   preferred_element_type=jnp.float32
                                   ) + b2_ref[...]


def _sc_gather_sum(src, tgt, ps, pt, E_pad):
    cpw = E_pad // (C * NW)
    mesh = plsc.VectorSubcoreMesh(core_axis_name="c", subcore_axis_name="s",
                                  num_cores=NC, num_subcores=NS)

    @functools.partial(
        pl.kernel,
        out_type=[jax.ShapeDtypeStruct((E_pad,), jnp.float32)
                  for _ in range(D16)],
        mesh=mesh,
        compiler_params=pltpu.CompilerParams(use_tc_tiling_on_sc=False,
                                             needs_layout_passes=False),
        scratch_types=[
            pltpu.VMEM((C,), jnp.int32),
            pltpu.VMEM((C,), jnp.int32),
            pltpu.VMEM((C, D16), jnp.float32),
            pltpu.VMEM((C, D16), jnp.float32),
            pltpu.VMEM((D16, C), jnp.float32),
            pltpu.SemaphoreType.DMA,
        ],
    )
    def sc_kernel(src_hbm, tgt_hbm, ps_hbm, pt_hbm, *rest):
        g_hbm = rest[:D16]
        idx_s, idx_t, rows_s, rows_t, colbuf, sem = rest[D16:]
        wid = lax.axis_index("s") * NC + lax.axis_index("c")
        lane = lax.iota(jnp.int32, D16)

        def chunk_body(k, carry):
            base = (wid * cpw + k) * C
            pltpu.sync_copy(src_hbm.at[pl.ds(base, C)], idx_s)
            pltpu.sync_copy(tgt_hbm.at[pl.ds(base, C)], idx_t)
            copies = []
            for j in range(SUB):
                sl = pl.ds(j * GR, GR)
                copies.append(
                    pltpu.async_copy(ps_hbm.at[idx_s.at[sl]], rows_s.at[sl],
                                     sem))
                copies.append(
                    pltpu.async_copy(pt_hbm.at[idx_t.at[sl]], rows_t.at[sl],
                                     sem))
            for cp in copies:
                cp.wait()

            def add_body(j, c2):
                h = rows_s[j] + rows_t[j]
                plsc.store_scatter(colbuf, [lane, jnp.full((D16,), j,
                                                           jnp.int32)], h)
                return c2

            lax.fori_loop(0, C, add_body, 0)
            for f in range(D16):
                pltpu.sync_copy(colbuf.at[f], g_hbm[f].at[pl.ds(base, C)])
            return carry

        lax.fori_loop(0, cpw, chunk_body, 0)

    return sc_kernel(src, tgt, ps, pt)


def kernel(x_s, x_t, edge_index, edge_attr, u, batch_e, W1, b1, W2, b2):
    n, f_xs = x_s.shape
    f_xt = x_t.shape[1]
    e, f_e = edge_attr.shape
    b, f_u = u.shape

    w1s = W1[:f_xs]
    w1t = W1[f_xs:f_xs + f_xt]
    w1e = W1[f_xs + f_xt:f_xs + f_xt + f_e]
    w1u = W1[f_xs + f_xt + f_e:]
    padc = lambda w: jnp.pad(w, ((0, 0), (0, D16 - f_e)))
    w1s_p, w1t_p, w1u_p, w1e_p = padc(w1s), padc(w1t), padc(w1u), padc(w1e)
    b1_col = jnp.pad(b1, (0, D16 - f_e)).reshape(D16, 1)
    w2_p = jnp.pad(W2, ((0, D16 - f_e), (0, 0)))
    b2_col = b2.reshape(f_e, 1)
    src = edge_index[0]
    tgt = edge_index[1]
    brow = batch_e.reshape(1, e)

    pst, ptt, put = pl.pallas_call(
        _prep_body,
        grid=(1,),
        in_specs=[
            pl.BlockSpec((f_xs, n), lambda i: (0, 0)),
            pl.BlockSpec((f_xt, n), lambda i: (0, 0)),
            pl.BlockSpec((f_u, b), lambda i: (0, 0)),
            pl.BlockSpec((f_xs, D16), lambda i: (0, 0)),
            pl.BlockSpec((f_xt, D16), lambda i: (0, 0)),
            pl.BlockSpec((f_u, D16), lambda i: (0, 0)),
            pl.BlockSpec((D16, 1), lambda i: (0, 0)),
        ],
        out_specs=[
            pl.BlockSpec((D16, n), lambda i: (0, 0)),
            pl.BlockSpec((D16, n), lambda i: (0, 0)),
            pl.BlockSpec((D16, b), lambda i: (0, 0)),
        ],
        out_shape=[
            jax.ShapeDtypeStruct((D16, n), jnp.float32),
            jax.ShapeDtypeStruct((D16, n), jnp.float32),
            jax.ShapeDtypeStruct((D16, b), jnp.float32),
        ],
    )(x_s.T, x_t.T, u.T, w1s_p, w1t_p, w1u_p, b1_col)
    ps = pst.T
    pt = ptt.T

    step = C * NW
    e_pad = ((e + step - 1) // step) * step
    src_p = jnp.pad(src, (0, e_pad - e))
    tgt_p = jnp.pad(tgt, (0, e_pad - e))
    g_feats = _sc_gather_sum(src_p, tgt_p, ps, pt, e_pad)

    grid_e = (e + BE - 1) // BE
    out_t = pl.pallas_call(
        _finish_body,
        grid=(grid_e,),
        in_specs=(
            [pl.BlockSpec((BE,), lambda i: (i,)) for _ in range(D16)]
            + [
                pl.BlockSpec((f_e, BE), lambda i: (0, i)),
                pl.BlockSpec((1, BE), lambda i: (0, i)),
                pl.BlockSpec((D16, b), lambda i: (0, 0)),
                pl.BlockSpec((f_e, D16), lambda i: (0, 0)),
                pl.BlockSpec((D16, f_e), lambda i: (0, 0)),
                pl.BlockSpec((f_e, 1), lambda i: (0, 0)),
            ]
        ),
        out_specs=pl.BlockSpec((f_e, BE), lambda i: (0, i)),
        out_shape=jax.ShapeDtypeStruct((f_e, e), jnp.float32),
    )(*g_feats, edge_attr.T, brow, put, w1e_p, w2_p, b2_col)
    return out_t.T


# speedup vs baseline: 9.2840x; 1.2078x over previous
import functools

import jax
import jax.numpy as jnp
from jax import lax
from jax.experimental import pallas as pl
from jax.experimental.pallas import tpu as pltpu
from jax.experimental.pallas import tpu_sc as plsc

D16 = 16
NC = 2
NS = 16
NW = NC * NS

C = 1024
GR = 128
SUB = C // GR

BE = 2048


def _prep_body(xst_ref, xtt_ref, ut_ref, w1s_ref, w1t_ref, w1u_ref, b1_ref,
               pst_ref, ptt_ref, put_ref):
    dims = (((0,), (0,)), ((), ()))
    pst_ref[...] = lax.dot_general(w1s_ref[...], xst_ref[...], dims,
                                   preferred_element_type=jnp.float32)
    ptt_ref[...] = lax.dot_general(w1t_ref[...], xtt_ref[...], dims,
                                   preferred_element_type=jnp.float32)
    put_ref[...] = lax.dot_general(w1u_ref[...], ut_ref[...], dims,
                                   preferred_element_type=jnp.float32
                                   ) + b1_ref[...]


def _finish_body(*refs):
    g_refs = refs[:D16]
    at_ref, brow_ref, put_ref, w1e_ref, w2_ref, b2_ref, out_ref = refs[D16:]
    be = at_ref.shape[1]
    nb = put_ref.shape[1]
    dims = (((0,), (0,)), ((), ()))
    a_term = lax.dot_general(w1e_ref[...], at_ref[...], dims,
                             preferred_element_type=jnp.float32)
    onehot = (brow_ref[...] == lax.broadcasted_iota(jnp.int32, (nb, be), 0)
              ).astype(jnp.float32)
    u_term = lax.dot_general(put_ref[...], onehot,
                             (((1,), (0,)), ((), ())),
                             preferred_element_type=jnp.float32)
    g = jnp.concatenate([r[...].reshape(1, be) for r in g_refs], axis=0)
    h = g + a_term + u_term
    h = jnp.where(h >= 0, h, 0.1 * h)
    out_ref[...] = lax.dot_general(w2_ref[...], h, dims,
                                   preferred_element_type=jnp.float32
                                   ) + b2_ref[...]


def _sc_gather_sum(src, tgt, ps, pt, E_pad):
    cpw = E_pad // (C * NW)
    mesh = plsc.VectorSubcoreMesh(core_axis_name="c", subcore_axis_name="s",
                                  num_cores=NC, num_subcores=NS)

    @functools.partial(
        pl.kernel,
        out_type=[jax.ShapeDtypeStruct((E_pad,), jnp.float32)
                  for _ in range(D16)],
        mesh=mesh,
        compiler_params=pltpu.CompilerParams(use_tc_tiling_on_sc=False,
                                             needs_layout_passes=False),
        scratch_types=[
            [pltpu.VMEM((C,), jnp.int32) for _ in range(2)],
            [pltpu.VMEM((C,), jnp.int32) for _ in range(2)],
            [pltpu.VMEM((C, D16), jnp.float32) for _ in range(2)],
            [pltpu.VMEM((C, D16), jnp.float32) for _ in range(2)],
            [pltpu.VMEM((D16, C), jnp.float32) for _ in range(2)],
            [pltpu.SemaphoreType.DMA for _ in range(2)],
            [pltpu.SemaphoreType.DMA for _ in range(2)],
            [pltpu.SemaphoreType.DMA for _ in range(2)],
        ],
    )
    def sc_kernel(src_hbm, tgt_hbm, ps_hbm, pt_hbm, *rest):
        g_hbm = rest[:D16]
        idx_s, idx_t, rows_s, rows_t, colbuf, isem, gsem, wsem = rest[D16:]
        wid = lax.axis_index("s") * NC + lax.axis_index("c")
        lane = lax.iota(jnp.int32, D16)
        kmax = cpw - 1

        def chunk_base(k):
            return (wid * cpw + jnp.minimum(k, kmax)) * C

        def issue_idx(k, b):
            base = chunk_base(k)
            pltpu.async_copy(src_hbm.at[pl.ds(base, C)], idx_s[b], isem[b])
            pltpu.async_copy(tgt_hbm.at[pl.ds(base, C)], idx_t[b], isem[b])

        def wait_idx(b):
            pltpu.make_async_copy(src_hbm.at[pl.ds(0, C)], idx_s[b],
                                  isem[b]).wait()
            pltpu.make_async_copy(tgt_hbm.at[pl.ds(0, C)], idx_t[b],
                                  isem[b]).wait()

        def issue_gathers(b):
            for j in range(SUB):
                sl = pl.ds(j * GR, GR)
                pltpu.async_copy(ps_hbm.at[idx_s[b].at[sl]],
                                 rows_s[b].at[sl], gsem[b])
                pltpu.async_copy(pt_hbm.at[idx_t[b].at[sl]],
                                 rows_t[b].at[sl], gsem[b])

        def wait_gathers(b):
            pltpu.make_async_copy(ps_hbm.at[pl.ds(0, C)], rows_s[b],
                                  gsem[b]).wait()
            pltpu.make_async_copy(pt_hbm.at[pl.ds(0, C)], rows_t[b],
                                  gsem[b]).wait()

        def issue_writes(k, b):
            base = chunk_base(k)
            for f in range(D16):
                pltpu.async_copy(colbuf[b].at[f], g_hbm[f].at[pl.ds(base, C)],
                                 wsem[b])

        def wait_writes(b):
            for f in range(D16):
                pltpu.make_async_copy(colbuf[b].at[f],
                                      g_hbm[f].at[pl.ds(0, C)],
                                      wsem[b]).wait()

        def compute(b):
            rs, rt, cb = rows_s[b], rows_t[b], colbuf[b]

            def add_body(j, c2):
                h = rs[j] + rt[j]
                plsc.store_scatter(cb, [lane, jnp.full((D16,), j,
                                                       jnp.int32)], h)
                return c2

            lax.fori_loop(0, C, add_body, 0)

        issue_idx(0, 0)
        issue_idx(1, 1)
        wait_idx(0)
        issue_gathers(0)

        def half(i, k, b):
            @pl.when(i > 0)
            def _():
                wait_writes(b)

            wait_gathers(b)
            issue_idx(k + 2, b)
            compute(b)
            issue_writes(k, b)
            wait_idx(b)
            issue_gathers(b)

        def pipe_body(i, carry):
            k0 = 2 * i
            wait_idx(1)
            issue_gathers(1)
            half(i, k0, 0)
            @pl.when(i > 0)
            def _():
                wait_writes(1)

            wait_gathers(1)
            issue_idx(k0 + 3, 1)
            compute(1)
            issue_writes(k0 + 1, 1)
            return carry

        lax.fori_loop(0, cpw // 2, pipe_body, 0)
        wait_gathers(0)
        wait_idx(1)
        wait_writes(0)
        wait_writes(1)

    return sc_kernel(src, tgt, ps, pt)


def kernel(x_s, x_t, edge_index, edge_attr, u, batch_e, W1, b1, W2, b2):
    n, f_xs = x_s.shape
    f_xt = x_t.shape[1]
    e, f_e = edge_attr.shape
    b, f_u = u.shape

    w1s = W1[:f_xs]
    w1t = W1[f_xs:f_xs + f_xt]
    w1e = W1[f_xs + f_xt:f_xs + f_xt + f_e]
    w1u = W1[f_xs + f_xt + f_e:]
    padc = lambda w: jnp.pad(w, ((0, 0), (0, D16 - f_e)))
    w1s_p, w1t_p, w1u_p, w1e_p = padc(w1s), padc(w1t), padc(w1u), padc(w1e)
    b1_col = jnp.pad(b1, (0, D16 - f_e)).reshape(D16, 1)
    w2_p = jnp.pad(W2, ((0, D16 - f_e), (0, 0)))
    b2_col = b2.reshape(f_e, 1)
    src = edge_index[0]
    tgt = edge_index[1]
    brow = batch_e.reshape(1, e)

    pst, ptt, put = pl.pallas_call(
        _prep_body,
        grid=(1,),
        in_specs=[
            pl.BlockSpec((f_xs, n), lambda i: (0, 0)),
            pl.BlockSpec((f_xt, n), lambda i: (0, 0)),
            pl.BlockSpec((f_u, b), lambda i: (0, 0)),
            pl.BlockSpec((f_xs, D16), lambda i: (0, 0)),
            pl.BlockSpec((f_xt, D16), lambda i: (0, 0)),
            pl.BlockSpec((f_u, D16), lambda i: (0, 0)),
            pl.BlockSpec((D16, 1), lambda i: (0, 0)),
        ],
        out_specs=[
            pl.BlockSpec((D16, n), lambda i: (0, 0)),
            pl.BlockSpec((D16, n), lambda i: (0, 0)),
            pl.BlockSpec((D16, b), lambda i: (0, 0)),
        ],
        out_shape=[
            jax.ShapeDtypeStruct((D16, n), jnp.float32),
            jax.ShapeDtypeStruct((D16, n), jnp.float32),
            jax.ShapeDtypeStruct((D16, b), jnp.float32),
        ],
    )(x_s.T, x_t.T, u.T, w1s_p, w1t_p, w1u_p, b1_col)
    ps = pst.T
    pt = ptt.T

    step = C * NW * 2
    e_pad = ((e + step - 1) // step) * step
    src_p = jnp.pad(src, (0, e_pad - e))
    tgt_p = jnp.pad(tgt, (0, e_pad - e))
    g_feats = _sc_gather_sum(src_p, tgt_p, ps, pt, e_pad)

    grid_e = (e + BE - 1) // BE
    out_t = pl.pallas_call(
        _finish_body,
        grid=(grid_e,),
        in_specs=(
            [pl.BlockSpec((BE,), lambda i: (i,)) for _ in range(D16)]
            + [
                pl.BlockSpec((f_e, BE), lambda i: (0, i)),
                pl.BlockSpec((1, BE), lambda i: (0, i)),
                pl.BlockSpec((D16, b), lambda i: (0, 0)),
                pl.BlockSpec((f_e, D16), lambda i: (0, 0)),
                pl.BlockSpec((D16, f_e), lambda i: (0, 0)),
                pl.BlockSpec((f_e, 1), lambda i: (0, 0)),
            ]
        ),
        out_specs=pl.BlockSpec((f_e, BE), lambda i: (0, i)),
        out_shape=jax.ShapeDtypeStruct((f_e, e), jnp.float32),
    )(*g_feats, edge_attr.T, brow, put, w1e_p, w2_p, b2_col)
    return out_t.T


# speedup vs baseline: 9.2847x; 1.0001x over previous
import functools

import jax
import jax.numpy as jnp
from jax import lax
from jax.experimental import pallas as pl
from jax.experimental.pallas import tpu as pltpu
from jax.experimental.pallas import tpu_sc as plsc

D16 = 16
NC = 2
NS = 16
NW = NC * NS

C = 1024
GR = 128
SUB = C // GR

BE = 2048


def _prep_body(xst_ref, xtt_ref, ut_ref, w1s_ref, w1t_ref, w1u_ref, b1_ref,
               pst_ref, ptt_ref, put_ref):
    dims = (((0,), (0,)), ((), ()))
    pst_ref[...] = lax.dot_general(w1s_ref[...], xst_ref[...], dims,
                                   preferred_element_type=jnp.float32)
    ptt_ref[...] = lax.dot_general(w1t_ref[...], xtt_ref[...], dims,
                                   preferred_element_type=jnp.float32)
    put_ref[...] = lax.dot_general(w1u_ref[...], ut_ref[...], dims,
                                   preferred_element_type=jnp.float32
                                   ) + b1_ref[...]


def _finish_body(*refs):
    g_refs = refs[:D16]
    at_ref, brow_ref, put_ref, w1e_ref, w2_ref, b2_ref, out_ref = refs[D16:]
    be = at_ref.shape[1]
    nb = put_ref.shape[1]
    dims = (((0,), (0,)), ((), ()))
    a_term = lax.dot_general(w1e_ref[...], at_ref[...], dims,
                             preferred_element_type=jnp.float32)
    onehot = (brow_ref[...] == lax.broadcasted_iota(jnp.int32, (nb, be), 0)
              ).astype(jnp.float32)
    u_term = lax.dot_general(put_ref[...], onehot,
                             (((1,), (0,)), ((), ())),
                             preferred_element_type=jnp.float32)
    g = jnp.concatenate([r[...].reshape(1, be) for r in g_refs], axis=0)
    h = g + a_term + u_term
    h = jnp.where(h >= 0, h, 0.1 * h)
    out_ref[...] = lax.dot_general(w2_ref[...], h, dims,
                                   preferred_element_type=jnp.float32
                                   ) + b2_ref[...]


def _sc_gather_sum(src, tgt, ps, pt, E_pad):
    cpw = E_pad // (C * NW)
    mesh = plsc.VectorSubcoreMesh(core_axis_name="c", subcore_axis_name="s",
                                  num_cores=NC, num_subcores=NS)

    @functools.partial(
        pl.kernel,
        out_type=[jax.ShapeDtypeStruct((E_pad,), jnp.float32)
                  for _ in range(D16)],
        mesh=mesh,
        compiler_params=pltpu.CompilerParams(use_tc_tiling_on_sc=False,
                                             needs_layout_passes=False),
        scratch_types=[
            [pltpu.VMEM((C,), jnp.int32) for _ in range(2)],
            [pltpu.VMEM((C,), jnp.int32) for _ in range(2)],
            [pltpu.VMEM((C, D16), jnp.float32) for _ in range(2)],
            [pltpu.VMEM((C, D16), jnp.float32) for _ in range(2)],
            [pltpu.VMEM((D16 * C,), jnp.float32) for _ in range(2)],
            [pltpu.SemaphoreType.DMA for _ in range(2)],
            [pltpu.SemaphoreType.DMA for _ in range(2)],
            [pltpu.SemaphoreType.DMA for _ in range(2)],
        ],
    )
    def sc_kernel(src_hbm, tgt_hbm, ps_hbm, pt_hbm, *rest):
        g_hbm = rest[:D16]
        idx_s, idx_t, rows_s, rows_t, colbuf, isem, gsem, wsem = rest[D16:]
        wid = lax.axis_index("s") * NC + lax.axis_index("c")
        lane = lax.iota(jnp.int32, D16)
        kmax = cpw - 1

        def chunk_base(k):
            return (wid * cpw + jnp.minimum(k, kmax)) * C

        def issue_idx(k, b):
            base = chunk_base(k)
            pltpu.async_copy(src_hbm.at[pl.ds(base, C)], idx_s[b], isem[b])
            pltpu.async_copy(tgt_hbm.at[pl.ds(base, C)], idx_t[b], isem[b])

        def wait_idx(b):
            pltpu.make_async_copy(src_hbm.at[pl.ds(0, C)], idx_s[b],
                                  isem[b]).wait()
            pltpu.make_async_copy(tgt_hbm.at[pl.ds(0, C)], idx_t[b],
                                  isem[b]).wait()

        def issue_gathers(b):
            for j in range(SUB):
                sl = pl.ds(j * GR, GR)
                pltpu.async_copy(ps_hbm.at[idx_s[b].at[sl]],
                                 rows_s[b].at[sl], gsem[b])
                pltpu.async_copy(pt_hbm.at[idx_t[b].at[sl]],
                                 rows_t[b].at[sl], gsem[b])

        def wait_gathers(b):
            pltpu.make_async_copy(ps_hbm.at[pl.ds(0, C)], rows_s[b],
                                  gsem[b]).wait()
            pltpu.make_async_copy(pt_hbm.at[pl.ds(0, C)], rows_t[b],
                                  gsem[b]).wait()

        def issue_writes(k, b):
            base = chunk_base(k)
            for f in range(D16):
                pltpu.async_copy(colbuf[b].at[pl.ds(f * C, C)],
                                 g_hbm[f].at[pl.ds(base, C)], wsem[b])

        def wait_writes(b):
            for f in range(D16):
                pltpu.make_async_copy(colbuf[b].at[pl.ds(f * C, C)],
                                      g_hbm[f].at[pl.ds(0, C)],
                                      wsem[b]).wait()

        UN = 16

        def compute(b):
            rs, rt, cbf = rows_s[b], rows_t[b], colbuf[b]

            def add_body(j, idxvec):
                for v in range(UN):
                    jj = j * UN + v
                    h = rs[jj] + rt[jj]
                    plsc.store_scatter(cbf, [idxvec], h)
                    idxvec = idxvec + 1
                return idxvec

            lax.fori_loop(0, C // UN, add_body, lane * C)

        issue_idx(0, 0)
        issue_idx(1, 1)
        wait_idx(0)
        issue_gathers(0)

        def half(i, k, b):
            @pl.when(i > 0)
            def _():
                wait_writes(b)

            wait_gathers(b)
            issue_idx(k + 2, b)
            compute(b)
            issue_writes(k, b)
            wait_idx(b)
            issue_gathers(b)

        def pipe_body(i, carry):
            k0 = 2 * i
            wait_idx(1)
            issue_gathers(1)
            half(i, k0, 0)
            @pl.when(i > 0)
            def _():
                wait_writes(1)

            wait_gathers(1)
            issue_idx(k0 + 3, 1)
            compute(1)
            issue_writes(k0 + 1, 1)
            return carry

        lax.fori_loop(0, cpw // 2, pipe_body, 0)
        wait_gathers(0)
        wait_idx(1)
        wait_writes(0)
        wait_writes(1)

    return sc_kernel(src, tgt, ps, pt)


def kernel(x_s, x_t, edge_index, edge_attr, u, batch_e, W1, b1, W2, b2):
    n, f_xs = x_s.shape
    f_xt = x_t.shape[1]
    e, f_e = edge_attr.shape
    b, f_u = u.shape

    w1s = W1[:f_xs]
    w1t = W1[f_xs:f_xs + f_xt]
    w1e = W1[f_xs + f_xt:f_xs + f_xt + f_e]
    w1u = W1[f_xs + f_xt + f_e:]
    padc = lambda w: jnp.pad(w, ((0, 0), (0, D16 - f_e)))
    w1s_p, w1t_p, w1u_p, w1e_p = padc(w1s), padc(w1t), padc(w1u), padc(w1e)
    b1_col = jnp.pad(b1, (0, D16 - f_e)).reshape(D16, 1)
    w2_p = jnp.pad(W2, ((0, D16 - f_e), (0, 0)))
    b2_col = b2.reshape(f_e, 1)
    src = edge_index[0]
    tgt = edge_index[1]
    brow = batch_e.reshape(1, e)

    pst, ptt, put = pl.pallas_call(
        _prep_body,
        grid=(1,),
        in_specs=[
            pl.BlockSpec((f_xs, n), lambda i: (0, 0)),
            pl.BlockSpec((f_xt, n), lambda i: (0, 0)),
            pl.BlockSpec((f_u, b), lambda i: (0, 0)),
            pl.BlockSpec((f_xs, D16), lambda i: (0, 0)),
            pl.BlockSpec((f_xt, D16), lambda i: (0, 0)),
            pl.BlockSpec((f_u, D16), lambda i: (0, 0)),
            pl.BlockSpec((D16, 1), lambda i: (0, 0)),
        ],
        out_specs=[
            pl.BlockSpec((D16, n), lambda i: (0, 0)),
            pl.BlockSpec((D16, n), lambda i: (0, 0)),
            pl.BlockSpec((D16, b), lambda i: (0, 0)),
        ],
        out_shape=[
            jax.ShapeDtypeStruct((D16, n), jnp.float32),
            jax.ShapeDtypeStruct((D16, n), jnp.float32),
            jax.ShapeDtypeStruct((D16, b), jnp.float32),
        ],
    )(x_s.T, x_t.T, u.T, w1s_p, w1t_p, w1u_p, b1_col)
    ps = pst.T
    pt = ptt.T

    step = C * NW * 2
    e_pad = ((e + step - 1) // step) * step
    src_p = jnp.pad(src, (0, e_pad - e))
    tgt_p = jnp.pad(tgt, (0, e_pad - e))
    g_feats = _sc_gather_sum(src_p, tgt_p, ps, pt, e_pad)

    grid_e = (e + BE - 1) // BE
    out_t = pl.pallas_call(
        _finish_body,
        grid=(grid_e,),
        in_specs=(
            [pl.BlockSpec((BE,), lambda i: (i,)) for _ in range(D16)]
            + [
                pl.BlockSpec((f_e, BE), lambda i: (0, i)),
                pl.BlockSpec((1, BE), lambda i: (0, i)),
                pl.BlockSpec((D16, b), lambda i: (0, 0)),
                pl.BlockSpec((f_e, D16), lambda i: (0, 0)),
                pl.BlockSpec((D16, f_e), lambda i: (0, 0)),
                pl.BlockSpec((f_e, 1), lambda i: (0, 0)),
            ]
        ),
        out_specs=pl.BlockSpec((f_e, BE), lambda i: (0, i)),
        out_shape=jax.ShapeDtypeStruct((f_e, e), jnp.float32),
    )(*g_feats, edge_attr.T, brow, put, w1e_p, w2_p, b2_col)
    return out_t.T
